# baseline (speedup 1.0000x reference)
import functools
import jax
import jax.numpy as jnp
from jax import lax
from jax.experimental import pallas as pl
from jax.experimental.pallas import tpu as pltpu
from jax.experimental.pallas import tpu_sc as plsc

N = 50000
D = 128
NE = 500000
SLAB = 32
NSLAB = 4

NC = 2
NS = 16
CH = 128
EPW = 16384
NE_PAD = EPW * NS * NC
NCHUNK = EPW // CH
RPW = 3200
N_SP = RPW * NS
LAST_ROWS = N - 15 * RPW

BN = 1000
GRID = N // BN


def _tc1_body(x_ref, w_ref, asr_ref, adr_ref,
              h_ref, av_s_ref, av_d_ref, c_ref):
    x = x_ref[:]
    w = w_ref[:]
    h = jnp.dot(x, w, preferred_element_type=jnp.float32)
    h_ref[0] = h[:, 0:32]
    h_ref[1] = h[:, 32:64]
    h_ref[2] = h[:, 64:96]
    h_ref[3] = h[:, 96:128]
    al_s = jnp.sum(h * asr_ref[:], axis=1, keepdims=True)
    al_d = jnp.sum(h * adr_ref[:], axis=1, keepdims=True)
    av_s_ref[:] = al_s
    av_d_ref[:] = al_d

    li = lax.broadcasted_iota(jnp.int32, (1, 8), 1)
    neg = jnp.full((1, 8), -3.0e38, jnp.float32)

    @pl.when(pl.program_id(0) == 0)
    def _init():
        c_ref[:] = neg

    m_s = jnp.max(al_s)
    m_d = jnp.max(al_d)
    upd = jnp.where(li == 0, m_s, jnp.where(li == 1, m_d, neg))
    c_ref[:] = jnp.maximum(c_ref[:], upd)

    @pl.when(pl.program_id(0) == GRID - 1)
    def _fin():
        cur = c_ref[:]
        s0 = jnp.max(jnp.where(li == 0, cur, neg))
        s1 = jnp.max(jnp.where(li == 1, cur, neg))
        c = jnp.maximum(s0 + s1, 0.0)
        c_ref[:] = jnp.where(li == 2, c, cur)


def _tc1(x, w, a_src, a_dst):
    return pl.pallas_call(
        _tc1_body,
        grid=(GRID,),
        in_specs=[
            pl.BlockSpec((BN, D), lambda i: (i, 0)),
            pl.BlockSpec((D, D), lambda i: (0, 0)),
            pl.BlockSpec((1, D), lambda i: (0, 0)),
            pl.BlockSpec((1, D), lambda i: (0, 0)),
        ],
        out_specs=[
            pl.BlockSpec((NSLAB, BN, SLAB), lambda i: (0, i, 0)),
            pl.BlockSpec((BN, 1), lambda i: (i, 0)),
            pl.BlockSpec((BN, 1), lambda i: (i, 0)),
            pl.BlockSpec((1, 8), lambda i: (0, 0)),
        ],
        out_shape=[
            jax.ShapeDtypeStruct((NSLAB, N, SLAB), jnp.float32),
            jax.ShapeDtypeStruct((N, 1), jnp.float32),
            jax.ShapeDtypeStruct((N, 1), jnp.float32),
            jax.ShapeDtypeStruct((1, 8), jnp.float32),
        ],
    )(x, w, a_src, a_dst)


def _make_sc():
    mesh = plsc.VectorSubcoreMesh(
        core_axis_name="c", subcore_axis_name="s",
        num_cores=NC, num_subcores=NS)

    def body(e0, e1, par, zrows, zden,
             as1, ad1, as2, ad2, h41, h42,
             acc1, acc2, den1, den2,
             vals_s, vals_d, src_c, dst_c, gidx_c, ex_c, rows, par_v,
             tabS_s, tabS_d, accS, denS, sem):
        cid = lax.axis_index("c")
        sid = lax.axis_index("s")
        row0 = sid * RPW

        pltpu.sync_copy(par, par_v)
        pltpu.sync_copy(zrows, accS.at[pl.ds(row0, RPW)])
        pltpu.sync_copy(zden, denS.at[pl.ds(row0, RPW)])
        plsc.subcore_barrier()

        for g in range(2):
            sref = e0 if g == 0 else e1
            dref = e1 if g == 0 else e0
            atab_s = as1 if g == 0 else as2
            atab_d = ad1 if g == 0 else ad2
            h4 = h41 if g == 0 else h42
            acc_out = acc1 if g == 0 else acc2
            den_out = den1 if g == 0 else den2

            @pl.when(sid == 0)
            def _ldtab(atab_s=atab_s, atab_d=atab_d):
                pltpu.sync_copy(atab_s, tabS_s)
                pltpu.sync_copy(atab_d, tabS_d)
            plsc.subcore_barrier()
            cvec = par_v[g]

            for fl in range(NSLAB):
                add_den = (fl == 0)
                f = (fl + 2 * cid) % NSLAB

                def chunk_body(ch, carry, add_den=add_den, sref=sref,
                               dref=dref, h4=h4, f=f, cvec=cvec):
                    base = (cid * NS + sid) * EPW + ch * CH
                    pltpu.sync_copy(sref.at[pl.ds(base, CH)], src_c)
                    pltpu.sync_copy(dref.at[pl.ds(base, CH)], dst_c)
                    off = f * N
                    for gq in range(CH // 16):
                        i_s = src_c[pl.ds(gq * 16, 16)]
                        gidx_c[pl.ds(gq * 16, 16)] = i_s + off
                    cp = pltpu.async_copy(h4.at[gidx_c], rows, sem)
                    pltpu.sync_copy(tabS_s.at[src_c], vals_s)
                    pltpu.sync_copy(tabS_d.at[dst_c], vals_d)
                    for gq in range(CH // 16):
                        a_s = vals_s[pl.ds(gq * 16, 16)]
                        a_d = vals_d[pl.ds(gq * 16, 16)]
                        e = a_s + a_d
                        e = jnp.where(e >= 0.0, e, 0.2 * e)
                        exv = jnp.exp(e - cvec)
                        gi = base + gq * 16 + lax.iota(jnp.int32, 16)
                        exv = jnp.where(gi < NE, exv, 0.0)
                        ex_c[pl.ds(gq * 16, 16)] = exv
                    if add_den:
                        pltpu.sync_copy(ex_c, denS.at[dst_c], add=True)
                    cp.wait()

                    def rgroup(q, c2):
                        ex16 = ex_c[pl.ds(q * 16, 16)]
                        for j in range(16):
                            r = q * 16 + j
                            exs = ex16[j]
                            rows[r, pl.ds(0, 16)] = rows[r, pl.ds(0, 16)] * exs
                            rows[r, pl.ds(16, 16)] = (
                                rows[r, pl.ds(16, 16)] * exs)
                        return c2
                    lax.fori_loop(0, CH // 16, rgroup, 0)
                    pltpu.sync_copy(rows, accS.at[dst_c], add=True)
                    return carry

                lax.fori_loop(0, NCHUNK, chunk_body, 0)
                plsc.subcore_barrier()

                obase = cid * (NSLAB * N) + f * N + row0

                @pl.when(sid < NS - 1)
                def _cp_full():
                    pltpu.sync_copy(accS.at[pl.ds(row0, RPW)],
                                    acc_out.at[pl.ds(obase, RPW)])

                @pl.when(sid == NS - 1)
                def _cp_last():
                    pltpu.sync_copy(accS.at[pl.ds(row0, LAST_ROWS)],
                                    acc_out.at[pl.ds(obase, LAST_ROWS)])

                if fl == 0:
                    dbase = cid * N + row0

                    @pl.when(sid < NS - 1)
                    def _cd_full():
                        pltpu.sync_copy(denS.at[pl.ds(row0, RPW)],
                                        den_out.at[pl.ds(dbase, RPW)])

                    @pl.when(sid == NS - 1)
                    def _cd_last():
                        pltpu.sync_copy(denS.at[pl.ds(row0, LAST_ROWS)],
                                        den_out.at[pl.ds(dbase, LAST_ROWS)])

                pltpu.sync_copy(zrows, accS.at[pl.ds(row0, RPW)])
                if g == 0 and fl == NSLAB - 1:
                    pltpu.sync_copy(zden, denS.at[pl.ds(row0, RPW)])
                plsc.subcore_barrier()

    return pl.kernel(
        body,
        out_type=[
            jax.ShapeDtypeStruct((NC * NSLAB * N, SLAB), jnp.float32),
            jax.ShapeDtypeStruct((NC * NSLAB * N, SLAB), jnp.float32),
            jax.ShapeDtypeStruct((NC * N,), jnp.float32),
            jax.ShapeDtypeStruct((NC * N,), jnp.float32),
        ],
        mesh=mesh,
        compiler_params=pltpu.CompilerParams(
            needs_layout_passes=False, use_tc_tiling_on_sc=False),
        scratch_types=[
            pltpu.VMEM((CH,), jnp.float32),
            pltpu.VMEM((CH,), jnp.float32),
            pltpu.VMEM((CH,), jnp.int32),
            pltpu.VMEM((CH,), jnp.int32),
            pltpu.VMEM((CH,), jnp.int32),
            pltpu.VMEM((CH,), jnp.float32),
            pltpu.VMEM((CH, SLAB), jnp.float32),
            pltpu.VMEM((2, 16), jnp.float32),
            pltpu.VMEM_SHARED((N,), jnp.float32),
            pltpu.VMEM_SHARED((N,), jnp.float32),
            pltpu.VMEM_SHARED((N_SP, SLAB), jnp.float32),
            pltpu.VMEM_SHARED((N_SP,), jnp.float32),
            pltpu.SemaphoreType.DMA,
        ],
    )


def _tc2_body(acc4_ref, den_ref, avs_ref, avd_ref, c_ref, h4_ref, x_ref,
              bg_ref, wl_ref, bl_ref, o_ref):
    acc = jnp.concatenate(
        [acc4_ref[0, 0] + acc4_ref[1, 0],
         acc4_ref[0, 1] + acc4_ref[1, 1],
         acc4_ref[0, 2] + acc4_ref[1, 2],
         acc4_ref[0, 3] + acc4_ref[1, 3]], axis=1)
    h = jnp.concatenate(
        [h4_ref[0], h4_ref[1], h4_ref[2], h4_ref[3]], axis=1)
    al_s = avs_ref[:]
    al_d = avd_ref[:]
    li = lax.broadcasted_iota(jnp.int32, (1, 8), 1)
    c = jnp.max(jnp.where(li == 2, c_ref[:], -3.0e38))
    s = al_s + al_d
    e = jnp.where(s >= 0.0, s, 0.2 * s)
    selfw = jnp.exp(e - c)
    denf = den_ref[0] + den_ref[1] + selfw
    accf = acc + selfw * h
    temp = accf / (denf + 1e-16) + bg_ref[:]
    y = temp + x_ref[:]
    o_ref[:] = jnp.dot(y, wl_ref[:],
                       preferred_element_type=jnp.float32) + bl_ref[:]


def _tc2(acc4, den, avs, avd, cpar, h4, x, bg, wl, bl):
    return pl.pallas_call(
        _tc2_body,
        grid=(GRID,),
        in_specs=[
            pl.BlockSpec((NC, NSLAB, BN, SLAB), lambda i: (0, 0, i, 0)),
            pl.BlockSpec((NC, BN, 1), lambda i: (0, i, 0)),
            pl.BlockSpec((BN, 1), lambda i: (i, 0)),
            pl.BlockSpec((BN, 1), lambda i: (i, 0)),
            pl.BlockSpec((1, 8), lambda i: (0, 0)),
            pl.BlockSpec((NSLAB, BN, SLAB), lambda i: (0, i, 0)),
            pl.BlockSpec((BN, D), lambda i: (i, 0)),
            pl.BlockSpec((1, D), lambda i: (0, 0)),
            pl.BlockSpec((D, D), lambda i: (0, 0)),
            pl.BlockSpec((1, D), lambda i: (0, 0)),
        ],
        out_specs=pl.BlockSpec((BN, D), lambda i: (i, 0)),
        out_shape=jax.ShapeDtypeStruct((N, D), jnp.float32),
    )(acc4, den, avs, avd, cpar, h4, x, bg, wl, bl)


def kernel(Xw, Xs, E, W1, a_src1, a_dst1, b1, W2, a_src2, a_dst2, b2,
           Wlw, blw):
    pad = jnp.zeros((NE_PAD - NE,), jnp.int32)
    e0 = jnp.concatenate([E[:, 0], pad])
    e1 = jnp.concatenate([E[:, 1], pad])
    del pad

    h41, avs1, avd1, cp1 = _tc1(Xs, W1, a_src1.reshape(1, D),
                                a_dst1.reshape(1, D))
    h42, avs2, avd2, cp2 = _tc1(Xw, W2, a_src2.reshape(1, D),
                                a_dst2.reshape(1, D))

    par = jnp.stack([
        jnp.broadcast_to(cp1[0, 2], (16,)),
        jnp.broadcast_to(cp2[0, 2], (16,)),
    ])

    zrows = jnp.zeros((RPW, SLAB), jnp.float32)
    zden = jnp.zeros((RPW,), jnp.float32)

    sc = _make_sc()
    acc1, acc2, den1, den2 = sc(
        e0, e1, par, zrows, zden,
        avs1.reshape(N), avd1.reshape(N), avs2.reshape(N), avd2.reshape(N),
        h41.reshape(NSLAB * N, SLAB), h42.reshape(NSLAB * N, SLAB))

    out1 = _tc2(acc1.reshape(NC, NSLAB, N, SLAB), den1.reshape(NC, N, 1),
                avs1, avd1, cp1, h41, Xw, b1.reshape(1, D), Wlw,
                blw.reshape(1, D))
    out2 = _tc2(acc2.reshape(NC, NSLAB, N, SLAB), den2.reshape(NC, N, 1),
                avs2, avd2, cp2, h42, Xs, b2.reshape(1, D), Wlw,
                blw.reshape(1, D))
    return (out1, out2)


# speedup vs baseline: 1.7141x; 1.7141x over previous
import jax
import jax.numpy as jnp
from jax import lax
from jax.experimental import pallas as pl
from jax.experimental.pallas import tpu as pltpu
from jax.experimental.pallas import tpu_sc as plsc

N = 50000
D = 128
NE = 500000
SLAB = 32
NSLAB = 4

NC = 2
NS = 16
CH = 128
EPW = 32768
NE_PAD = EPW * NS
NROW = NE_PAD // CH
KB = 16
NBLK = EPW // (KB * CH)
RPW = 3200
N_SP = RPW * NS
LAST_ROWS = N - 15 * RPW

BN = 1000
GRID = N // BN


def _tc1_body(x_ref, w_ref, asr_ref, adr_ref,
              h_ref, av_s_ref, av_d_ref, c_ref):
    x = x_ref[:]
    w = w_ref[:]
    h = jnp.dot(x, w, preferred_element_type=jnp.float32)
    h_ref[0] = h[:, 0:32]
    h_ref[1] = h[:, 32:64]
    h_ref[2] = h[:, 64:96]
    h_ref[3] = h[:, 96:128]
    al_s = jnp.sum(h * asr_ref[:], axis=1, keepdims=True)
    al_d = jnp.sum(h * adr_ref[:], axis=1, keepdims=True)
    av_s_ref[:] = al_s
    av_d_ref[:] = al_d

    li = lax.broadcasted_iota(jnp.int32, (1, 8), 1)
    neg = jnp.full((1, 8), -3.0e38, jnp.float32)

    @pl.when(pl.program_id(0) == 0)
    def _init():
        c_ref[:] = neg

    m_s = jnp.max(al_s)
    m_d = jnp.max(al_d)
    upd = jnp.where(li == 0, m_s, jnp.where(li == 1, m_d, neg))
    c_ref[:] = jnp.maximum(c_ref[:], upd)

    @pl.when(pl.program_id(0) == GRID - 1)
    def _fin():
        cur = c_ref[:]
        s0 = jnp.max(jnp.where(li == 0, cur, neg))
        s1 = jnp.max(jnp.where(li == 1, cur, neg))
        c = jnp.maximum(s0 + s1, 0.0)
        c_ref[:] = jnp.where(li == 2, c, cur)


def _tc1(x, w, a_src, a_dst):
    return pl.pallas_call(
        _tc1_body,
        grid=(GRID,),
        in_specs=[
            pl.BlockSpec((BN, D), lambda i: (i, 0)),
            pl.BlockSpec((D, D), lambda i: (0, 0)),
            pl.BlockSpec((1, D), lambda i: (0, 0)),
            pl.BlockSpec((1, D), lambda i: (0, 0)),
        ],
        out_specs=[
            pl.BlockSpec((NSLAB, BN, SLAB), lambda i: (0, i, 0)),
            pl.BlockSpec((BN, 1), lambda i: (i, 0)),
            pl.BlockSpec((BN, 1), lambda i: (i, 0)),
            pl.BlockSpec((1, 8), lambda i: (0, 0)),
        ],
        out_shape=[
            jax.ShapeDtypeStruct((NSLAB, N, SLAB), jnp.float32),
            jax.ShapeDtypeStruct((N, 1), jnp.float32),
            jax.ShapeDtypeStruct((N, 1), jnp.float32),
            jax.ShapeDtypeStruct((1, 8), jnp.float32),
        ],
    )(x, w, a_src, a_dst)


def _make_sc():
    mesh = plsc.VectorSubcoreMesh(
        core_axis_name="c", subcore_axis_name="s",
        num_cores=NC, num_subcores=NS)

    def body(e0, e1, par, zrows, zden,
             as1, ad1, as2, ad2, h41, h42,
             acc1, acc2, den1, den2,
             sblk_a, sblk_b, dblk_a, dblk_b,
             vs_a, vd_a, vs_b, vd_b,
             gidx_a, gidx_b, ex_a, ex_b, rows_a, rows_b, par_v,
             tabS_s, tabS_d, accS, denS,
             semB, semTa, semTb, semRa, semRb, semS):
        cid = lax.axis_index("c")
        sid = lax.axis_index("s")
        row0 = sid * RPW

        pltpu.sync_copy(par, par_v)
        pltpu.sync_copy(zrows, accS.at[pl.ds(row0, RPW)])
        pltpu.sync_copy(zden, denS.at[pl.ds(row0, RPW)])
        plsc.subcore_barrier()

        def wait_blk(sb, db):
            pltpu.make_async_copy(e0.at[pl.ds(0, KB)], sb, semB).wait()
            pltpu.make_async_copy(e0.at[pl.ds(0, KB)], db, semB).wait()

        for g in range(2):
            sref = e0 if g == 0 else e1
            dref = e1 if g == 0 else e0
            atab_s = as1 if g == 0 else as2
            atab_d = ad1 if g == 0 else ad2
            h4 = h41 if g == 0 else h42
            acc_out = acc1 if g == 0 else acc2
            den_out = den1 if g == 0 else den2
            den_cid = g

            @pl.when(sid == 0)
            def _ldtab(atab_s=atab_s, atab_d=atab_d):
                pltpu.sync_copy(atab_s, tabS_s)
                pltpu.sync_copy(atab_d, tabS_d)
            plsc.subcore_barrier()
            cvec = par_v[g]

            for fl in range(2):
                add_den = (fl == 0)
                f = cid * 2 + fl
                off = f * N

                def issue_blk(bi, sb, db, sref=sref, dref=dref):
                    r0 = sid * (EPW // CH) + bi * KB
                    pltpu.async_copy(sref.at[pl.ds(r0, KB)], sb, semB)
                    pltpu.async_copy(dref.at[pl.ds(r0, KB)], db, semB)

                def half_chunk(bi, k, sblk, dblk, gidx, vs, vd,
                               h4=h4):
                    t1 = pltpu.async_copy(tabS_s.at[sblk.at[k]], vs, semTa)
                    t2 = pltpu.async_copy(tabS_d.at[dblk.at[k]], vd, semTa)
                    for gq in range(CH // 16):
                        i_s = sblk[k, pl.ds(gq * 16, 16)]
                        gidx[pl.ds(gq * 16, 16)] = i_s + off
                    rg = pltpu.async_copy(h4.at[gidx], rows_a, semRa)
                    return t1, t2, rg

                def finish_chunk(bi, k, dblk, vs, vd, ex_c, rows, t1, t2, rg,
                                 add_den=add_den, den_cid=den_cid,
                                 cvec=cvec):
                    t1.wait()
                    t2.wait()
                    base = sid * EPW + bi * (KB * CH) + k * CH
                    for gq in range(CH // 16):
                        a_s = vs[pl.ds(gq * 16, 16)]
                        a_d = vd[pl.ds(gq * 16, 16)]
                        e = a_s + a_d
                        e = jnp.where(e >= 0.0, e, 0.2 * e)
                        exv = jnp.exp(e - cvec)
                        gi = base + gq * 16 + lax.iota(jnp.int32, 16)
                        exv = jnp.where(gi < NE, exv, 0.0)
                        ex_c[pl.ds(gq * 16, 16)] = exv
                    if add_den:
                        @pl.when(cid == den_cid)
                        def _den():
                            pltpu.async_copy(ex_c, denS.at[dblk.at[k]],
                                             semS, add=True)
                    rg.wait()

                    def rgroup(q, c2):
                        ex16 = ex_c[pl.ds(q * 16, 16)]
                        for j in range(16):
                            r = q * 16 + j
                            exs = ex16[j]
                            rows[r, pl.ds(0, 16)] = (
                                rows[r, pl.ds(0, 16)] * exs)
                            rows[r, pl.ds(16, 16)] = (
                                rows[r, pl.ds(16, 16)] * exs)
                        return c2
                    lax.fori_loop(0, CH // 16, rgroup, 0)
                    pltpu.async_copy(rows, accS.at[dblk.at[k]],
                                     semS, add=True)

                def drain_pair(add_den=add_den, den_cid=den_cid):
                    pltpu.make_async_copy(
                        rows_a, accS.at[pl.ds(0, CH)], semS).wait()
                    pltpu.make_async_copy(
                        rows_b, accS.at[pl.ds(0, CH)], semS).wait()
                    if add_den:
                        @pl.when(cid == den_cid)
                        def _dw():
                            pltpu.make_async_copy(
                                ex_a, denS.at[pl.ds(0, CH)], semS).wait()
                            pltpu.make_async_copy(
                                ex_b, denS.at[pl.ds(0, CH)], semS).wait()

                def process_block(bi, sblk, dblk):
                    def pair(p, carry, bi=bi, sblk=sblk, dblk=dblk):
                        ka = p * 2
                        kb = p * 2 + 1
                        ha = half_chunk(bi, ka, sblk, dblk, gidx_a,
                                        vs_a, vd_a)
                        t1b = pltpu.async_copy(
                            tabS_s.at[sblk.at[kb]], vs_b, semTb)
                        t2b = pltpu.async_copy(
                            tabS_d.at[dblk.at[kb]], vd_b, semTb)
                        for gq in range(CH // 16):
                            i_s = sblk[kb, pl.ds(gq * 16, 16)]
                            gidx_b[pl.ds(gq * 16, 16)] = i_s + off
                        rgb = pltpu.async_copy(h4.at[gidx_b], rows_b, semRb)
                        finish_chunk(bi, ka, dblk, vs_a, vd_a, ex_a,
                                     rows_a, *ha)
                        finish_chunk(bi, kb, dblk, vs_b, vd_b, ex_b,
                                     rows_b, t1b, t2b, rgb)
                        drain_pair()
                        return carry
                    lax.fori_loop(0, KB // 2, pair, 0)

                issue_blk(0, sblk_a, dblk_a)

                def blk_loop(b2, carry):
                    b0 = b2 * 2
                    wait_blk(sblk_a, dblk_a)
                    issue_blk(b0 + 1, sblk_b, dblk_b)
                    process_block(b0, sblk_a, dblk_a)
                    wait_blk(sblk_b, dblk_b)

                    @pl.when(b2 < NBLK // 2 - 1)
                    def _nxt():
                        issue_blk(b0 + 2, sblk_a, dblk_a)
                    process_block(b0 + 1, sblk_b, dblk_b)
                    return carry
                lax.fori_loop(0, NBLK // 2, blk_loop, 0)
                plsc.subcore_barrier()

                obase = f * N + row0

                @pl.when(sid < NS - 1)
                def _cp_full():
                    pltpu.sync_copy(accS.at[pl.ds(row0, RPW)],
                                    acc_out.at[pl.ds(obase, RPW)])

                @pl.when(sid == NS - 1)
                def _cp_last():
                    pltpu.sync_copy(accS.at[pl.ds(row0, LAST_ROWS)],
                                    acc_out.at[pl.ds(obase, LAST_ROWS)])

                if fl == 0:
                    @pl.when(cid == den_cid)
                    def _cp_den(den_out=den_out):
                        @pl.when(sid < NS - 1)
                        def _cd_full():
                            pltpu.sync_copy(denS.at[pl.ds(row0, RPW)],
                                            den_out.at[pl.ds(row0, RPW)])

                        @pl.when(sid == NS - 1)
                        def _cd_last():
                            pltpu.sync_copy(
                                denS.at[pl.ds(row0, LAST_ROWS)],
                                den_out.at[pl.ds(row0, LAST_ROWS)])

                pltpu.sync_copy(zrows, accS.at[pl.ds(row0, RPW)])
                plsc.subcore_barrier()

    return pl.kernel(
        body,
        out_type=[
            jax.ShapeDtypeStruct((NSLAB * N, SLAB), jnp.float32),
            jax.ShapeDtypeStruct((NSLAB * N, SLAB), jnp.float32),
            jax.ShapeDtypeStruct((N,), jnp.float32),
            jax.ShapeDtypeStruct((N,), jnp.float32),
        ],
        mesh=mesh,
        compiler_params=pltpu.CompilerParams(
            needs_layout_passes=False, use_tc_tiling_on_sc=False),
        scratch_types=[
            pltpu.VMEM((KB, CH), jnp.int32),
            pltpu.VMEM((KB, CH), jnp.int32),
            pltpu.VMEM((KB, CH), jnp.int32),
            pltpu.VMEM((KB, CH), jnp.int32),
            pltpu.VMEM((CH,), jnp.float32),
            pltpu.VMEM((CH,), jnp.float32),
            pltpu.VMEM((CH,), jnp.float32),
            pltpu.VMEM((CH,), jnp.float32),
            pltpu.VMEM((CH,), jnp.int32),
            pltpu.VMEM((CH,), jnp.int32),
            pltpu.VMEM((CH,), jnp.float32),
            pltpu.VMEM((CH,), jnp.float32),
            pltpu.VMEM((CH, SLAB), jnp.float32),
            pltpu.VMEM((CH, SLAB), jnp.float32),
            pltpu.VMEM((2, 16), jnp.float32),
            pltpu.VMEM_SHARED((N,), jnp.float32),
            pltpu.VMEM_SHARED((N,), jnp.float32),
            pltpu.VMEM_SHARED((N_SP, SLAB), jnp.float32),
            pltpu.VMEM_SHARED((N_SP,), jnp.float32),
            pltpu.SemaphoreType.DMA,
            pltpu.SemaphoreType.DMA,
            pltpu.SemaphoreType.DMA,
            pltpu.SemaphoreType.DMA,
            pltpu.SemaphoreType.DMA,
            pltpu.SemaphoreType.DMA,
        ],
    )


def _tc2_body(acc4_ref, den_ref, avs_ref, avd_ref, c_ref, h4_ref, x_ref,
              bg_ref, wl_ref, bl_ref, o_ref):
    acc = jnp.concatenate(
        [acc4_ref[0], acc4_ref[1], acc4_ref[2], acc4_ref[3]], axis=1)
    h = jnp.concatenate(
        [h4_ref[0], h4_ref[1], h4_ref[2], h4_ref[3]], axis=1)
    al_s = avs_ref[:]
    al_d = avd_ref[:]
    li = lax.broadcasted_iota(jnp.int32, (1, 8), 1)
    c = jnp.max(jnp.where(li == 2, c_ref[:], -3.0e38))
    s = al_s + al_d
    e = jnp.where(s >= 0.0, s, 0.2 * s)
    selfw = jnp.exp(e - c)
    denf = den_ref[:] + selfw
    accf = acc + selfw * h
    temp = accf / (denf + 1e-16) + bg_ref[:]
    y = temp + x_ref[:]
    o_ref[:] = jnp.dot(y, wl_ref[:],
                       preferred_element_type=jnp.float32) + bl_ref[:]


def _tc2(acc4, den, avs, avd, cpar, h4, x, bg, wl, bl):
    return pl.pallas_call(
        _tc2_body,
        grid=(GRID,),
        in_specs=[
            pl.BlockSpec((NSLAB, BN, SLAB), lambda i: (0, i, 0)),
            pl.BlockSpec((BN, 1), lambda i: (i, 0)),
            pl.BlockSpec((BN, 1), lambda i: (i, 0)),
            pl.BlockSpec((BN, 1), lambda i: (i, 0)),
            pl.BlockSpec((1, 8), lambda i: (0, 0)),
            pl.BlockSpec((NSLAB, BN, SLAB), lambda i: (0, i, 0)),
            pl.BlockSpec((BN, D), lambda i: (i, 0)),
            pl.BlockSpec((1, D), lambda i: (0, 0)),
            pl.BlockSpec((D, D), lambda i: (0, 0)),
            pl.BlockSpec((1, D), lambda i: (0, 0)),
        ],
        out_specs=pl.BlockSpec((BN, D), lambda i: (i, 0)),
        out_shape=jax.ShapeDtypeStruct((N, D), jnp.float32),
    )(acc4, den, avs, avd, cpar, h4, x, bg, wl, bl)


def kernel(Xw, Xs, E, W1, a_src1, a_dst1, b1, W2, a_src2, a_dst2, b2,
           Wlw, blw):
    pad = jnp.zeros((NE_PAD - NE,), jnp.int32)
    e0 = jnp.concatenate([E[:, 0], pad]).reshape(NROW, CH)
    e1 = jnp.concatenate([E[:, 1], pad]).reshape(NROW, CH)

    h41, avs1, avd1, cp1 = _tc1(Xs, W1, a_src1.reshape(1, D),
                                a_dst1.reshape(1, D))
    h42, avs2, avd2, cp2 = _tc1(Xw, W2, a_src2.reshape(1, D),
                                a_dst2.reshape(1, D))

    par = jnp.stack([
        jnp.broadcast_to(cp1[0, 2], (16,)),
        jnp.broadcast_to(cp2[0, 2], (16,)),
    ])

    zrows = jnp.zeros((RPW, SLAB), jnp.float32)
    zden = jnp.zeros((RPW,), jnp.float32)

    sc = _make_sc()
    acc1, acc2, den1, den2 = sc(
        e0, e1, par, zrows, zden,
        avs1.reshape(N), avd1.reshape(N), avs2.reshape(N), avd2.reshape(N),
        h41.reshape(NSLAB * N, SLAB), h42.reshape(NSLAB * N, SLAB))

    out1 = _tc2(acc1.reshape(NSLAB, N, SLAB), den1.reshape(N, 1),
                avs1, avd1, cp1, h41, Xw, b1.reshape(1, D), Wlw,
                blw.reshape(1, D))
    out2 = _tc2(acc2.reshape(NSLAB, N, SLAB), den2.reshape(N, 1),
                avs2, avd2, cp2, h42, Xs, b2.reshape(1, D), Wlw,
                blw.reshape(1, D))
    return (out1, out2)


# speedup vs baseline: 1.9867x; 1.1590x over previous
import jax
import jax.numpy as jnp
from jax import lax
from jax.experimental import pallas as pl
from jax.experimental.pallas import tpu as pltpu
from jax.experimental.pallas import tpu_sc as plsc

N = 50000
D = 128
NE = 500000
SLAB = 32
NSLAB = 4

NC = 2
NS = 16
CH = 96
EPW = 32256
NE_PAD = EPW * NS
NROW = NE_PAD // CH
KB = 8
NBLK = EPW // (KB * CH)
RPW = 3200
N_SP = RPW * NS
LAST_ROWS = N - 15 * RPW

BN = 1000
GRID = N // BN


def _tc1_body(x_ref, w_ref, asr_ref, adr_ref,
              h_ref, av_s_ref, av_d_ref, c_ref):
    x = x_ref[:]
    w = w_ref[:]
    h = jnp.dot(x, w, preferred_element_type=jnp.float32)
    h_ref[0] = h[:, 0:32]
    h_ref[1] = h[:, 32:64]
    h_ref[2] = h[:, 64:96]
    h_ref[3] = h[:, 96:128]
    al_s = jnp.sum(h * asr_ref[:], axis=1, keepdims=True)
    al_d = jnp.sum(h * adr_ref[:], axis=1, keepdims=True)
    av_s_ref[:] = al_s
    av_d_ref[:] = al_d

    li = lax.broadcasted_iota(jnp.int32, (1, 8), 1)
    neg = jnp.full((1, 8), -3.0e38, jnp.float32)

    @pl.when(pl.program_id(0) == 0)
    def _init():
        c_ref[:] = neg

    m_s = jnp.max(al_s)
    m_d = jnp.max(al_d)
    upd = jnp.where(li == 0, m_s, jnp.where(li == 1, m_d, neg))
    c_ref[:] = jnp.maximum(c_ref[:], upd)

    @pl.when(pl.program_id(0) == GRID - 1)
    def _fin():
        cur = c_ref[:]
        s0 = jnp.max(jnp.where(li == 0, cur, neg))
        s1 = jnp.max(jnp.where(li == 1, cur, neg))
        c = jnp.maximum(s0 + s1, 0.0)
        c_ref[:] = jnp.where(li == 2, c, cur)


def _tc1(x, w, a_src, a_dst):
    return pl.pallas_call(
        _tc1_body,
        grid=(GRID,),
        in_specs=[
            pl.BlockSpec((BN, D), lambda i: (i, 0)),
            pl.BlockSpec((D, D), lambda i: (0, 0)),
            pl.BlockSpec((1, D), lambda i: (0, 0)),
            pl.BlockSpec((1, D), lambda i: (0, 0)),
        ],
        out_specs=[
            pl.BlockSpec((NSLAB, BN, SLAB), lambda i: (0, i, 0)),
            pl.BlockSpec((BN, 1), lambda i: (i, 0)),
            pl.BlockSpec((BN, 1), lambda i: (i, 0)),
            pl.BlockSpec((1, 8), lambda i: (0, 0)),
        ],
        out_shape=[
            jax.ShapeDtypeStruct((NSLAB, N, SLAB), jnp.float32),
            jax.ShapeDtypeStruct((N, 1), jnp.float32),
            jax.ShapeDtypeStruct((N, 1), jnp.float32),
            jax.ShapeDtypeStruct((1, 8), jnp.float32),
        ],
    )(x, w, a_src, a_dst)


def _make_sc():
    mesh = plsc.VectorSubcoreMesh(
        core_axis_name="c", subcore_axis_name="s",
        num_cores=NC, num_subcores=NS)

    def body(e0, e1, par, zrows, zden,
             as1, ad1, as2, ad2, h41, h42,
             acc1, acc2, den1, den2,
             sblk_a, sblk_b, dblk_a, dblk_b,
             vs_a, vd_a, vs_b, vd_b,
             gidx_a, gidx_b, ex_a, ex_b, ex_c2, ex_d,
             rows_a, rows_b, rows_c, rows_d, par_v,
             tabS_s, tabS_d, accS, denS,
             semB, semTa, semTb, semRa, semRb, semS0, semS1):
        cid = lax.axis_index("c")
        sid = lax.axis_index("s")
        row0 = sid * RPW

        pltpu.sync_copy(par, par_v)
        pltpu.sync_copy(zrows, accS.at[pl.ds(row0, RPW)])
        pltpu.sync_copy(zden, denS.at[pl.ds(row0, RPW)])
        plsc.subcore_barrier()

        def wait_blk(sb, db):
            pltpu.make_async_copy(e0.at[pl.ds(0, KB)], sb, semB).wait()
            pltpu.make_async_copy(e0.at[pl.ds(0, KB)], db, semB).wait()

        for g in range(2):
            sref = e0 if g == 0 else e1
            dref = e1 if g == 0 else e0
            atab_s = as1 if g == 0 else as2
            atab_d = ad1 if g == 0 else ad2
            h4 = h41 if g == 0 else h42
            acc_out = acc1 if g == 0 else acc2
            den_out = den1 if g == 0 else den2
            den_cid = g

            @pl.when(sid == 0)
            def _ldtab(atab_s=atab_s, atab_d=atab_d):
                pltpu.sync_copy(atab_s, tabS_s)
                pltpu.sync_copy(atab_d, tabS_d)
            plsc.subcore_barrier()
            cvec = par_v[g]

            for fl in range(2):
                add_den = (fl == 0)
                f = cid * 2 + fl
                off = f * N

                def issue_blk(bi, sb, db, sref=sref, dref=dref):
                    r0 = sid * (EPW // CH) + bi * KB
                    pltpu.async_copy(sref.at[pl.ds(r0, KB)], sb, semB)
                    pltpu.async_copy(dref.at[pl.ds(r0, KB)], db, semB)

                def issue_chunk(k, sblk, dblk, gidx, vs, vd, rows,
                                semT, semR, h4=h4):
                    t1 = pltpu.async_copy(tabS_s.at[sblk.at[k]], vs, semT)
                    t2 = pltpu.async_copy(tabS_d.at[dblk.at[k]], vd, semT)
                    for gq in range(CH // 16):
                        i_s = sblk[k, pl.ds(gq * 16, 16)]
                        gidx[pl.ds(gq * 16, 16)] = i_s + off
                    rg = pltpu.async_copy(h4.at[gidx], rows, semR)
                    return t1, t2, rg

                def finish_chunk(bi, k, dblk, vs, vd, ex_c, rows, semS,
                                 t1, t2, rg, add_den=add_den,
                                 den_cid=den_cid, cvec=cvec):
                    t1.wait()
                    t2.wait()
                    base = sid * EPW + bi * (KB * CH) + k * CH
                    for gq in range(CH // 16):
                        a_s = vs[pl.ds(gq * 16, 16)]
                        a_d = vd[pl.ds(gq * 16, 16)]
                        e = a_s + a_d
                        e = jnp.where(e >= 0.0, e, 0.2 * e)
                        exv = jnp.exp(e - cvec)
                        gi = base + gq * 16 + lax.iota(jnp.int32, 16)
                        exv = jnp.where(gi < NE, exv, 0.0)
                        ex_c[pl.ds(gq * 16, 16)] = exv
                    if add_den:
                        @pl.when(cid == den_cid)
                        def _den():
                            pltpu.async_copy(ex_c, denS.at[dblk.at[k]],
                                             semS, add=True)
                    rg.wait()

                    def rgroup(q, c2):
                        ex16 = ex_c[pl.ds(q * 16, 16)]
                        for j in range(16):
                            r = q * 16 + j
                            exs = ex16[j]
                            rows[r, pl.ds(0, 16)] = (
                                rows[r, pl.ds(0, 16)] * exs)
                            rows[r, pl.ds(16, 16)] = (
                                rows[r, pl.ds(16, 16)] * exs)
                        return c2
                    lax.fori_loop(0, CH // 16, rgroup, 0)
                    pltpu.async_copy(rows, accS.at[dblk.at[k]],
                                     semS, add=True)

                def drain_set(rowsA, rowsB, semS, add_den=add_den,
                              den_cid=den_cid):
                    pltpu.make_async_copy(
                        rowsA, accS.at[pl.ds(0, CH)], semS).wait()
                    pltpu.make_async_copy(
                        rowsB, accS.at[pl.ds(0, CH)], semS).wait()
                    if add_den:
                        @pl.when(cid == den_cid)
                        def _dw():
                            pltpu.make_async_copy(
                                ex_a, denS.at[pl.ds(0, CH)], semS).wait()
                            pltpu.make_async_copy(
                                ex_b, denS.at[pl.ds(0, CH)], semS).wait()

                def pair_body(bi, ka, sblk, dblk, exA, exB, rowsA, rowsB,
                              semS):
                    kb = ka + 1
                    fa = issue_chunk(ka, sblk, dblk, gidx_a, vs_a, vd_a,
                                     rowsA, semTa, semRa)
                    fb = issue_chunk(kb, sblk, dblk, gidx_b, vs_b, vd_b,
                                     rowsB, semTb, semRb)
                    finish_chunk(bi, ka, dblk, vs_a, vd_a, exA, rowsA,
                                 semS, *fa)
                    finish_chunk(bi, kb, dblk, vs_b, vd_b, exB, rowsB,
                                 semS, *fb)

                def process_block(bi, sblk, dblk, first):
                    def quad(q, carry, bi=bi, sblk=sblk, dblk=dblk,
                             first=first):
                        do_drain = jnp.logical_or(q > 0,
                                                  jnp.logical_not(first))

                        @pl.when(do_drain)
                        def _d0():
                            drain_set(rows_a, rows_b, semS0)
                        pair_body(bi, q * 4, sblk, dblk, ex_a, ex_b,
                                  rows_a, rows_b, semS0)

                        @pl.when(do_drain)
                        def _d1():
                            drain_set(rows_c, rows_d, semS1)
                        pair_body(bi, q * 4 + 2, sblk, dblk, ex_c2, ex_d,
                                  rows_c, rows_d, semS1)
                        return carry
                    lax.fori_loop(0, KB // 4, quad, 0)

                issue_blk(0, sblk_a, dblk_a)

                def blk_loop(b2, carry):
                    b0 = b2 * 2
                    wait_blk(sblk_a, dblk_a)
                    issue_blk(b0 + 1, sblk_b, dblk_b)
                    process_block(b0, sblk_a, dblk_a, b2 == 0)
                    wait_blk(sblk_b, dblk_b)

                    @pl.when(b2 < NBLK // 2 - 1)
                    def _nxt():
                        issue_blk(b0 + 2, sblk_a, dblk_a)
                    process_block(b0 + 1, sblk_b, dblk_b, False)
                    return carry
                lax.fori_loop(0, NBLK // 2, blk_loop, 0)
                drain_set(rows_a, rows_b, semS0)
                drain_set(rows_c, rows_d, semS1)
                plsc.subcore_barrier()

                obase = f * N + row0

                @pl.when(sid < NS - 1)
                def _cp_full():
                    pltpu.sync_copy(accS.at[pl.ds(row0, RPW)],
                                    acc_out.at[pl.ds(obase, RPW)])

                @pl.when(sid == NS - 1)
                def _cp_last():
                    pltpu.sync_copy(accS.at[pl.ds(row0, LAST_ROWS)],
                                    acc_out.at[pl.ds(obase, LAST_ROWS)])

                if fl == 0:
                    @pl.when(cid == den_cid)
                    def _cp_den(den_out=den_out):
                        @pl.when(sid < NS - 1)
                        def _cd_full():
                            pltpu.sync_copy(denS.at[pl.ds(row0, RPW)],
                                            den_out.at[pl.ds(row0, RPW)])

                        @pl.when(sid == NS - 1)
                        def _cd_last():
                            pltpu.sync_copy(
                                denS.at[pl.ds(row0, LAST_ROWS)],
                                den_out.at[pl.ds(row0, LAST_ROWS)])

                pltpu.sync_copy(zrows, accS.at[pl.ds(row0, RPW)])
                plsc.subcore_barrier()

    return pl.kernel(
        body,
        out_type=[
            jax.ShapeDtypeStruct((NSLAB * N, SLAB), jnp.float32),
            jax.ShapeDtypeStruct((NSLAB * N, SLAB), jnp.float32),
            jax.ShapeDtypeStruct((N,), jnp.float32),
            jax.ShapeDtypeStruct((N,), jnp.float32),
        ],
        mesh=mesh,
        compiler_params=pltpu.CompilerParams(
            needs_layout_passes=False, use_tc_tiling_on_sc=False),
        scratch_types=[
            pltpu.VMEM((KB, CH), jnp.int32),
            pltpu.VMEM((KB, CH), jnp.int32),
            pltpu.VMEM((KB, CH), jnp.int32),
            pltpu.VMEM((KB, CH), jnp.int32),
            pltpu.VMEM((CH,), jnp.float32),
            pltpu.VMEM((CH,), jnp.float32),
            pltpu.VMEM((CH,), jnp.float32),
            pltpu.VMEM((CH,), jnp.float32),
            pltpu.VMEM((CH,), jnp.int32),
            pltpu.VMEM((CH,), jnp.int32),
            pltpu.VMEM((CH,), jnp.float32),
            pltpu.VMEM((CH,), jnp.float32),
            pltpu.VMEM((CH,), jnp.float32),
            pltpu.VMEM((CH,), jnp.float32),
            pltpu.VMEM((CH, SLAB), jnp.float32),
            pltpu.VMEM((CH, SLAB), jnp.float32),
            pltpu.VMEM((CH, SLAB), jnp.float32),
            pltpu.VMEM((CH, SLAB), jnp.float32),
            pltpu.VMEM((2, 16), jnp.float32),
            pltpu.VMEM_SHARED((N,), jnp.float32),
            pltpu.VMEM_SHARED((N,), jnp.float32),
            pltpu.VMEM_SHARED((N_SP, SLAB), jnp.float32),
            pltpu.VMEM_SHARED((N_SP,), jnp.float32),
            pltpu.SemaphoreType.DMA,
            pltpu.SemaphoreType.DMA,
            pltpu.SemaphoreType.DMA,
            pltpu.SemaphoreType.DMA,
            pltpu.SemaphoreType.DMA,
            pltpu.SemaphoreType.DMA,
            pltpu.SemaphoreType.DMA,
        ],
    )


def _tc2_body(acc4_ref, den_ref, avs_ref, avd_ref, c_ref, h4_ref, x_ref,
              bg_ref, wl_ref, bl_ref, o_ref):
    acc = jnp.concatenate(
        [acc4_ref[0], acc4_ref[1], acc4_ref[2], acc4_ref[3]], axis=1)
    h = jnp.concatenate(
        [h4_ref[0], h4_ref[1], h4_ref[2], h4_ref[3]], axis=1)
    al_s = avs_ref[:]
    al_d = avd_ref[:]
    li = lax.broadcasted_iota(jnp.int32, (1, 8), 1)
    c = jnp.max(jnp.where(li == 2, c_ref[:], -3.0e38))
    s = al_s + al_d
    e = jnp.where(s >= 0.0, s, 0.2 * s)
    selfw = jnp.exp(e - c)
    denf = den_ref[:] + selfw
    accf = acc + selfw * h
    temp = accf / (denf + 1e-16) + bg_ref[:]
    y = temp + x_ref[:]
    o_ref[:] = jnp.dot(y, wl_ref[:],
                       preferred_element_type=jnp.float32) + bl_ref[:]


def _tc2(acc4, den, avs, avd, cpar, h4, x, bg, wl, bl):
    return pl.pallas_call(
        _tc2_body,
        grid=(GRID,),
        in_specs=[
            pl.BlockSpec((NSLAB, BN, SLAB), lambda i: (0, i, 0)),
            pl.BlockSpec((BN, 1), lambda i: (i, 0)),
            pl.BlockSpec((BN, 1), lambda i: (i, 0)),
            pl.BlockSpec((BN, 1), lambda i: (i, 0)),
            pl.BlockSpec((1, 8), lambda i: (0, 0)),
            pl.BlockSpec((NSLAB, BN, SLAB), lambda i: (0, i, 0)),
            pl.BlockSpec((BN, D), lambda i: (i, 0)),
            pl.BlockSpec((1, D), lambda i: (0, 0)),
            pl.BlockSpec((D, D), lambda i: (0, 0)),
            pl.BlockSpec((1, D), lambda i: (0, 0)),
        ],
        out_specs=pl.BlockSpec((BN, D), lambda i: (i, 0)),
        out_shape=jax.ShapeDtypeStruct((N, D), jnp.float32),
    )(acc4, den, avs, avd, cpar, h4, x, bg, wl, bl)


def kernel(Xw, Xs, E, W1, a_src1, a_dst1, b1, W2, a_src2, a_dst2, b2,
           Wlw, blw):
    pad = jnp.zeros((NE_PAD - NE,), jnp.int32)
    e0 = jnp.concatenate([E[:, 0], pad]).reshape(NROW, CH)
    e1 = jnp.concatenate([E[:, 1], pad]).reshape(NROW, CH)

    h41, avs1, avd1, cp1 = _tc1(Xs, W1, a_src1.reshape(1, D),
                                a_dst1.reshape(1, D))
    h42, avs2, avd2, cp2 = _tc1(Xw, W2, a_src2.reshape(1, D),
                                a_dst2.reshape(1, D))

    par = jnp.stack([
        jnp.broadcast_to(cp1[0, 2], (16,)),
        jnp.broadcast_to(cp2[0, 2], (16,)),
    ])

    zrows = jnp.zeros((RPW, SLAB), jnp.float32)
    zden = jnp.zeros((RPW,), jnp.float32)

    sc = _make_sc()
    acc1, acc2, den1, den2 = sc(
        e0, e1, par, zrows, zden,
        avs1.reshape(N), avd1.reshape(N), avs2.reshape(N), avd2.reshape(N),
        h41.reshape(NSLAB * N, SLAB), h42.reshape(NSLAB * N, SLAB))

    out1 = _tc2(acc1.reshape(NSLAB, N, SLAB), den1.reshape(N, 1),
                avs1, avd1, cp1, h41, Xw, b1.reshape(1, D), Wlw,
                blw.reshape(1, D))
    out2 = _tc2(acc2.reshape(NSLAB, N, SLAB), den2.reshape(N, 1),
                avs2, avd2, cp2, h42, Xs, b2.reshape(1, D), Wlw,
                blw.reshape(1, D))
    return (out1, out2)
